# Initial kernel scaffold; baseline (speedup 1.0000x reference)
import jax
import jax.numpy as jnp
from jax.experimental import pallas as pl


def kernel(x, adj, W1, b1, W2, b2):
    raise NotImplementedError("write your pallas kernel here")


# speedup vs baseline: 12.4390x; 12.4390x over previous
import functools

import jax
import jax.numpy as jnp
from jax import lax
from jax.experimental import pallas as pl
from jax.experimental.pallas import tpu as pltpu
from jax.experimental.pallas import tpu_sc as plsc

_NC = 2
_NS = 16
_NW = _NC * _NS
_K = 80


def _sc_degree_hist(dst, ones, n_nodes):
    e = dst.shape[0]
    epw = e // _NW
    nb = epw // _K
    rpt = n_nodes // _NS
    mesh = plsc.VectorSubcoreMesh(core_axis_name="c", subcore_axis_name="s")

    @functools.partial(
        pl.kernel,
        out_type=jax.ShapeDtypeStruct((_NC * n_nodes, 16), jnp.float32),
        mesh=mesh,
        scratch_types=[
            pltpu.VMEM((_K,), jnp.int32),
            pltpu.VMEM((_K, 16), jnp.float32),
            pltpu.VMEM_SHARED((n_nodes, 16), jnp.float32),
        ],
    )
    def k(dst_hbm, ones_hbm, out_hbm, idx_v, ones_v, acc):
        cid = lax.axis_index("c")
        sid = lax.axis_index("s")
        wid = sid * _NC + cid
        base = wid * epw
        row0 = sid * rpt
        pltpu.sync_copy(ones_hbm.at[pl.ds(row0, rpt)], acc.at[pl.ds(row0, rpt)])
        pltpu.sync_copy(ones_hbm.at[pl.ds(0, _K)], ones_v)
        plsc.subcore_barrier()

        def body(b, carry):
            pltpu.sync_copy(dst_hbm.at[pl.ds(base + b * _K, _K)], idx_v)
            pltpu.sync_copy(ones_v, acc.at[idx_v], add=True)
            return carry

        lax.fori_loop(0, nb, body, 0)
        plsc.subcore_barrier()
        pltpu.sync_copy(acc.at[pl.ds(row0, rpt)],
                        out_hbm.at[pl.ds(cid * n_nodes + row0, rpt)])

    return k(dst, ones).reshape(_NC, n_nodes, 16)


def _sc_edge_agg(g, src, dst):
    n_nodes, feat = g.shape
    e = src.shape[0]
    epw = e // _NW
    nb = epw // _K
    rpt = n_nodes // _NS
    mesh = plsc.VectorSubcoreMesh(core_axis_name="c", subcore_axis_name="s")

    @functools.partial(
        pl.kernel,
        out_type=jax.ShapeDtypeStruct((_NC * n_nodes, feat), jnp.float32),
        mesh=mesh,
        scratch_types=[
            pltpu.VMEM((_K,), jnp.int32),
            pltpu.VMEM((_K,), jnp.int32),
            pltpu.VMEM((_K, feat), jnp.float32),
            pltpu.VMEM_SHARED((n_nodes, feat), jnp.float32),
            pltpu.SemaphoreType.DMA,
        ],
    )
    def k(g_hbm, src_hbm, dst_hbm, out_hbm, src_v, dst_v, rows_v, acc, sem):
        cid = lax.axis_index("c")
        sid = lax.axis_index("s")
        wid = sid * _NC + cid
        base = wid * epw
        row0 = sid * rpt
        pltpu.sync_copy(g_hbm.at[pl.ds(row0, rpt)], acc.at[pl.ds(row0, rpt)])
        plsc.subcore_barrier()

        def body(b, carry):
            off = base + b * _K
            pltpu.sync_copy(src_hbm.at[pl.ds(off, _K)], src_v)
            pltpu.sync_copy(dst_hbm.at[pl.ds(off, _K)], dst_v)
            pltpu.async_copy(g_hbm.at[src_v], rows_v, sem).wait()
            pltpu.sync_copy(rows_v, acc.at[dst_v], add=True)
            return carry

        lax.fori_loop(0, nb, body, 0)
        plsc.subcore_barrier()
        pltpu.sync_copy(acc.at[pl.ds(row0, rpt)],
                        out_hbm.at[pl.ds(cid * n_nodes + row0, rpt)])

    return k(g, src, dst).reshape(_NC, n_nodes, feat)


_BR = 640


def _tc_layer1(x, W1, p0, p1):
    n, fin = x.shape
    h = W1.shape[1]

    def body(x_ref, w_ref, p0_ref, p1_ref, g_ref, s_ref):
        deg = p0_ref[:, 0:1] + p1_ref[:, 0:1] - 1.0
        s = lax.rsqrt(deg)
        hv = jnp.dot(x_ref[...], w_ref[...], preferred_element_type=jnp.float32)
        g_ref[...] = s * hv
        s_ref[...] = jnp.broadcast_to(s, (_BR, 16))

    return pl.pallas_call(
        body,
        grid=(n // _BR,),
        in_specs=[
            pl.BlockSpec((_BR, fin), lambda i: (i, 0)),
            pl.BlockSpec((fin, h), lambda i: (0, 0)),
            pl.BlockSpec((_BR, 16), lambda i: (i, 0)),
            pl.BlockSpec((_BR, 16), lambda i: (i, 0)),
        ],
        out_specs=[
            pl.BlockSpec((_BR, h), lambda i: (i, 0)),
            pl.BlockSpec((_BR, 16), lambda i: (i, 0)),
        ],
        out_shape=[
            jax.ShapeDtypeStruct((n, h), jnp.float32),
            jax.ShapeDtypeStruct((n, 16), jnp.float32),
        ],
    )(x, W1, p0, p1)


def _tc_layer2(a0, a1, g1, s, W2, b1):
    n, h = g1.shape
    c = W2.shape[1]

    def body(a0_ref, a1_ref, g1_ref, s_ref, w_ref, b_ref, out_ref):
        sv = s_ref[:, 0:1]
        pre = sv * (a0_ref[...] + a1_ref[...] - g1_ref[...]) + b_ref[...]
        hv = jnp.maximum(pre, 0.0)
        h2 = jnp.dot(hv, w_ref[...], preferred_element_type=jnp.float32)
        out_ref[...] = sv * h2

    return pl.pallas_call(
        body,
        grid=(n // _BR,),
        in_specs=[
            pl.BlockSpec((_BR, h), lambda i: (i, 0)),
            pl.BlockSpec((_BR, h), lambda i: (i, 0)),
            pl.BlockSpec((_BR, h), lambda i: (i, 0)),
            pl.BlockSpec((_BR, 16), lambda i: (i, 0)),
            pl.BlockSpec((h, c), lambda i: (0, 0)),
            pl.BlockSpec((1, h), lambda i: (0, 0)),
        ],
        out_specs=pl.BlockSpec((_BR, c), lambda i: (i, 0)),
        out_shape=jax.ShapeDtypeStruct((n, c), jnp.float32),
    )(a0, a1, g1, s, W2, b1)


def _tc_out(c0, c1, g2, s, b2):
    n, cp = g2.shape
    c = b2.shape[1]

    def body(c0_ref, c1_ref, g2_ref, s_ref, b_ref, out_ref):
        sv = s_ref[:, 0:1]
        zc0 = c0_ref[:, 0:c]
        zc1 = c1_ref[:, 0:c]
        zg2 = g2_ref[:, 0:c]
        z = sv * (zc0 + zc1 - zg2) + b_ref[...]
        z = z - jnp.max(z, axis=1, keepdims=True)
        out_ref[...] = z - jnp.log(jnp.sum(jnp.exp(z), axis=1, keepdims=True))

    return pl.pallas_call(
        body,
        grid=(n // _BR,),
        in_specs=[
            pl.BlockSpec((_BR, cp), lambda i: (i, 0)),
            pl.BlockSpec((_BR, cp), lambda i: (i, 0)),
            pl.BlockSpec((_BR, cp), lambda i: (i, 0)),
            pl.BlockSpec((_BR, 16), lambda i: (i, 0)),
            pl.BlockSpec((1, c), lambda i: (0, 0)),
        ],
        out_specs=pl.BlockSpec((_BR, c), lambda i: (i, 0)),
        out_shape=jax.ShapeDtypeStruct((n, c), jnp.float32),
    )(c0, c1, g2, s, b2)


def kernel(x, adj, W1, b1, W2, b2):
    n = x.shape[0]
    n_pad = -(-n // 1280) * 1280
    xp = jnp.pad(x, ((0, n_pad - n), (0, 0)))
    src = adj[0].astype(jnp.int32)
    dst = adj[1].astype(jnp.int32)
    ones = jnp.ones((n_pad, 16), jnp.float32)

    ncls = W2.shape[1]
    W2p = jnp.pad(W2, ((0, 0), (0, 128 - ncls)))

    p = _sc_degree_hist(dst, ones, n_pad)
    g1, s = _tc_layer1(xp, W1, p[0], p[1])
    a = _sc_edge_agg(g1, src, dst)
    g2 = _tc_layer2(a[0], a[1], g1, s, W2p, b1.reshape(1, -1))
    c = _sc_edge_agg(g2, src, dst)
    return _tc_out(c[0], c[1], g2, s, b2.reshape(1, -1))[:n]


# speedup vs baseline: 29.5193x; 2.3731x over previous
import functools

import jax
import jax.numpy as jnp
from jax import lax
from jax.experimental import pallas as pl
from jax.experimental.pallas import tpu as pltpu
from jax.experimental.pallas import tpu_sc as plsc

_NC = 2
_NS = 16
_NW = _NC * _NS
_K = 40
_D = 5
_W = 8


def _sc_degree_hist(dst3, ones, n_nodes):
    nb = dst3.shape[1]
    rpt = n_nodes // _NS
    mesh = plsc.VectorSubcoreMesh(core_axis_name="c", subcore_axis_name="s")

    @functools.partial(
        pl.kernel,
        out_type=jax.ShapeDtypeStruct((_NC * n_nodes, 16), jnp.float32),
        mesh=mesh,
        scratch_types=[
            pltpu.VMEM((nb, _K), jnp.int32),
            pltpu.VMEM((_K, 16), jnp.float32),
            pltpu.VMEM_SHARED((n_nodes, 16), jnp.float32),
            pltpu.SemaphoreType.DMA,
        ],
    )
    def k(dst_hbm, ones_hbm, out_hbm, idx_v, ones_v, acc, sem):
        cid = lax.axis_index("c")
        sid = lax.axis_index("s")
        wid = sid * _NC + cid
        row0 = sid * rpt
        pltpu.sync_copy(dst_hbm.at[wid], idx_v)
        pltpu.sync_copy(ones_hbm.at[pl.ds(row0, rpt)], acc.at[pl.ds(row0, rpt)])
        pltpu.sync_copy(ones_hbm.at[pl.ds(0, _K)], ones_v)
        plsc.subcore_barrier()

        def fire(b, carry):
            pltpu.async_copy(ones_v, acc.at[idx_v.at[b]], sem, add=True)
            return carry

        def fire_and_wait(b, carry):
            pltpu.async_copy(ones_v, acc.at[idx_v.at[b]], sem, add=True)
            pltpu.make_async_copy(ones_v, acc.at[idx_v.at[0]], sem).wait()
            return carry

        def drain(b, carry):
            pltpu.make_async_copy(ones_v, acc.at[idx_v.at[0]], sem).wait()
            return carry

        lax.fori_loop(0, _W, fire, 0)
        lax.fori_loop(_W, nb, fire_and_wait, 0)
        lax.fori_loop(0, _W, drain, 0)
        plsc.subcore_barrier()
        pltpu.sync_copy(acc.at[pl.ds(row0, rpt)],
                        out_hbm.at[pl.ds(cid * n_nodes + row0, rpt)])

    return k(dst3, ones).reshape(_NC, n_nodes, 16)


def _sc_edge_agg(g, src3, dst3):
    n_nodes, feat = g.shape
    nb = src3.shape[1]
    rpt = n_nodes // _NS
    mesh = plsc.VectorSubcoreMesh(core_axis_name="c", subcore_axis_name="s")

    d2 = 2 * _D
    assert nb % d2 == 0 and nb >= 2 * d2

    @functools.partial(
        pl.kernel,
        out_type=jax.ShapeDtypeStruct((_NC * n_nodes, feat), jnp.float32),
        mesh=mesh,
        scratch_types=[
            pltpu.VMEM((d2, _K), jnp.int32),
            pltpu.VMEM((d2, _K), jnp.int32),
            pltpu.VMEM((_D * _K, feat), jnp.float32),
            pltpu.VMEM_SHARED((n_nodes, feat), jnp.float32),
        ] + [pltpu.SemaphoreType.DMA] * (5 * _D),
    )
    def k(g_hbm, src_hbm, dst_hbm, out_hbm, src_v, dst_v, rows_v, acc, *sems):
        gsem = sems[:_D]
        ssem = sems[_D:3 * _D]
        dsem = sems[3 * _D:5 * _D]
        cid = lax.axis_index("c")
        sid = lax.axis_index("s")
        wid = sid * _NC + cid
        row0 = sid * rpt
        pltpu.sync_copy(g_hbm.at[pl.ds(row0, rpt)], acc.at[pl.ds(row0, rpt)])
        plsc.subcore_barrier()

        def slot(d):
            return rows_v.at[pl.ds(d * _K, _K)]

        def fire_idx(b, e):
            pltpu.async_copy(src_hbm.at[wid, b], src_v.at[e], ssem[e])
            pltpu.async_copy(dst_hbm.at[wid, b], dst_v.at[e], dsem[e])

        def wait_sidx(b, e):
            pltpu.make_async_copy(
                src_hbm.at[wid, b], src_v.at[e], ssem[e]).wait()

        def fire_gather(b, f, d):
            pltpu.async_copy(g_hbm.at[src_v.at[f]], slot(d), gsem[d])

        def scatter(b, e, d):
            pltpu.make_async_copy(
                g_hbm.at[src_v.at[e]], slot(d), gsem[d]).wait()
            pltpu.make_async_copy(
                dst_hbm.at[wid, b], dst_v.at[e], dsem[e]).wait()
            pltpu.sync_copy(slot(d), acc.at[dst_v.at[e]], add=True)

        for e in range(d2):
            fire_idx(e, e)
        for d in range(_D):
            wait_sidx(d, d)
            fire_gather(d, d, d)

        def outer(gi, carry):
            for j in range(d2):
                b = gi * d2 + j
                d = j % _D
                f = (j + _D) % d2
                scatter(b, j, d)
                fire_idx(b + d2, j)
                wait_sidx(b + _D, f)
                fire_gather(b + _D, f, d)
            return carry

        lax.fori_loop(0, nb // d2 - 1, outer, 0)
        for j in range(_D):
            b = nb - d2 + j
            d = j % _D
            f = (j + _D) % d2
            scatter(b, j, d)
            wait_sidx(b + _D, f)
            fire_gather(b + _D, f, d)
        for j in range(_D, d2):
            scatter(nb - d2 + j, j, j % _D)

        plsc.subcore_barrier()
        pltpu.sync_copy(acc.at[pl.ds(row0, rpt)],
                        out_hbm.at[pl.ds(cid * n_nodes + row0, rpt)])

    return k(g, src3, dst3).reshape(_NC, n_nodes, feat)


_BR = 640


def _tc_layer1(x, W1, p0, p1):
    n, fin = x.shape
    h = W1.shape[1]

    def body(x_ref, w_ref, p0_ref, p1_ref, g_ref, s_ref):
        deg = p0_ref[:, 0:1] + p1_ref[:, 0:1] - 1.0
        s = lax.rsqrt(deg)
        hv = jnp.dot(x_ref[...], w_ref[...], preferred_element_type=jnp.float32)
        g_ref[...] = s * hv
        s_ref[...] = jnp.broadcast_to(s, (_BR, 16))

    return pl.pallas_call(
        body,
        grid=(n // _BR,),
        in_specs=[
            pl.BlockSpec((_BR, fin), lambda i: (i, 0)),
            pl.BlockSpec((fin, h), lambda i: (0, 0)),
            pl.BlockSpec((_BR, 16), lambda i: (i, 0)),
            pl.BlockSpec((_BR, 16), lambda i: (i, 0)),
        ],
        out_specs=[
            pl.BlockSpec((_BR, h), lambda i: (i, 0)),
            pl.BlockSpec((_BR, 16), lambda i: (i, 0)),
        ],
        out_shape=[
            jax.ShapeDtypeStruct((n, h), jnp.float32),
            jax.ShapeDtypeStruct((n, 16), jnp.float32),
        ],
    )(x, W1, p0, p1)


def _tc_layer2(a0, a1, g1, s, W2, b1):
    n, h = g1.shape
    c = W2.shape[1]

    def body(a0_ref, a1_ref, g1_ref, s_ref, w_ref, b_ref, out_ref):
        sv = s_ref[:, 0:1]
        pre = sv * (a0_ref[...] + a1_ref[...] - g1_ref[...]) + b_ref[...]
        hv = jnp.maximum(pre, 0.0)
        h2 = jnp.dot(hv, w_ref[...], preferred_element_type=jnp.float32)
        out_ref[...] = sv * h2

    return pl.pallas_call(
        body,
        grid=(n // _BR,),
        in_specs=[
            pl.BlockSpec((_BR, h), lambda i: (i, 0)),
            pl.BlockSpec((_BR, h), lambda i: (i, 0)),
            pl.BlockSpec((_BR, h), lambda i: (i, 0)),
            pl.BlockSpec((_BR, 16), lambda i: (i, 0)),
            pl.BlockSpec((h, c), lambda i: (0, 0)),
            pl.BlockSpec((1, h), lambda i: (0, 0)),
        ],
        out_specs=pl.BlockSpec((_BR, c), lambda i: (i, 0)),
        out_shape=jax.ShapeDtypeStruct((n, c), jnp.float32),
    )(a0, a1, g1, s, W2, b1)


def _tc_out(c0, c1, g2, s, b2):
    n, cp = g2.shape
    c = b2.shape[1]

    def body(c0_ref, c1_ref, g2_ref, s_ref, b_ref, out_ref):
        sv = s_ref[:, 0:1]
        zc0 = c0_ref[:, 0:c]
        zc1 = c1_ref[:, 0:c]
        zg2 = g2_ref[:, 0:c]
        z = sv * (zc0 + zc1 - zg2) + b_ref[...]
        z = z - jnp.max(z, axis=1, keepdims=True)
        out_ref[...] = z - jnp.log(jnp.sum(jnp.exp(z), axis=1, keepdims=True))

    return pl.pallas_call(
        body,
        grid=(n // _BR,),
        in_specs=[
            pl.BlockSpec((_BR, cp), lambda i: (i, 0)),
            pl.BlockSpec((_BR, cp), lambda i: (i, 0)),
            pl.BlockSpec((_BR, cp), lambda i: (i, 0)),
            pl.BlockSpec((_BR, 16), lambda i: (i, 0)),
            pl.BlockSpec((1, c), lambda i: (0, 0)),
        ],
        out_specs=pl.BlockSpec((_BR, c), lambda i: (i, 0)),
        out_shape=jax.ShapeDtypeStruct((n, c), jnp.float32),
    )(c0, c1, g2, s, b2)


def kernel(x, adj, W1, b1, W2, b2):
    n = x.shape[0]
    n_pad = -(-n // 1280) * 1280
    xp = jnp.pad(x, ((0, n_pad - n), (0, 0)))
    e = adj.shape[1]
    nb = e // _NW // _K
    src = adj[0].astype(jnp.int32).reshape(_NW, nb, _K)
    dst = adj[1].astype(jnp.int32).reshape(_NW, nb, _K)
    ones = jnp.ones((n_pad, 16), jnp.float32)
    ncls = W2.shape[1]
    W2p = jnp.pad(W2, ((0, 0), (0, 128 - ncls)))

    p = _sc_degree_hist(dst, ones, n_pad)
    g1, s = _tc_layer1(xp, W1, p[0], p[1])
    a = _sc_edge_agg(g1, src, dst)
    g2 = _tc_layer2(a[0], a[1], g1, s, W2p, b1.reshape(1, -1))
    c = _sc_edge_agg(g2, src, dst)
    return _tc_out(c[0], c[1], g2, s, b2.reshape(1, -1))[:n]


# speedup vs baseline: 36.1877x; 1.2259x over previous
import functools

import jax
import jax.numpy as jnp
from jax import lax
from jax.experimental import pallas as pl
from jax.experimental.pallas import tpu as pltpu
from jax.experimental.pallas import tpu_sc as plsc

_NC = 2
_NS = 16
_NW = _NC * _NS
_K = 40
_D = 5
_W = 8


def _sc_degree_hist(adjr, n_nodes):
    nb = adjr.shape[2]
    rpt = n_nodes // _NS
    mesh = plsc.VectorSubcoreMesh(core_axis_name="c", subcore_axis_name="s")

    @functools.partial(
        pl.kernel,
        out_type=[jax.ShapeDtypeStruct((n_nodes, 16), jnp.float32),
                  jax.ShapeDtypeStruct((n_nodes, 16), jnp.float32)],
        mesh=mesh,
        scratch_types=[
            pltpu.VMEM((nb, _K), jnp.int32),
            pltpu.VMEM((_K, 16), jnp.float32),
            pltpu.VMEM_SHARED((n_nodes, 16), jnp.float32),
            pltpu.SemaphoreType.DMA,
        ],
    )
    def k(adj_hbm, p0_hbm, p1_hbm, idx_v, ones_v, acc, sem):
        cid = lax.axis_index("c")
        sid = lax.axis_index("s")
        wid = sid * _NC + cid
        row0 = sid * rpt
        pltpu.sync_copy(adj_hbm.at[1, wid], idx_v)

        def setone(r, carry):
            ones_v[r, :] = jnp.full((16,), 1.0, jnp.float32)
            return carry

        lax.fori_loop(0, _K, setone, 0)

        def init(j, carry):
            pltpu.sync_copy(ones_v, acc.at[pl.ds(row0 + j * _K, _K)])
            return carry

        lax.fori_loop(0, rpt // _K, init, 0)
        plsc.subcore_barrier()

        def fire(b, carry):
            pltpu.async_copy(ones_v, acc.at[idx_v.at[b]], sem, add=True)
            return carry

        def fire_and_wait(b, carry):
            pltpu.async_copy(ones_v, acc.at[idx_v.at[b]], sem, add=True)
            pltpu.make_async_copy(ones_v, acc.at[idx_v.at[0]], sem).wait()
            return carry

        def drain(b, carry):
            pltpu.make_async_copy(ones_v, acc.at[idx_v.at[0]], sem).wait()
            return carry

        lax.fori_loop(0, _W, fire, 0)
        lax.fori_loop(_W, nb, fire_and_wait, 0)
        lax.fori_loop(0, _W, drain, 0)
        plsc.subcore_barrier()

        @pl.when(cid == 0)
        def _():
            pltpu.sync_copy(acc.at[pl.ds(row0, rpt)],
                            p0_hbm.at[pl.ds(row0, rpt)])

        @pl.when(cid == 1)
        def _():
            pltpu.sync_copy(acc.at[pl.ds(row0, rpt)],
                            p1_hbm.at[pl.ds(row0, rpt)])

    return k(adjr)


def _sc_edge_agg(g, adjr):
    n_nodes, feat = g.shape
    nb = adjr.shape[2]
    rpt = n_nodes // _NS
    mesh = plsc.VectorSubcoreMesh(core_axis_name="c", subcore_axis_name="s")
    d2 = 2 * _D
    assert nb % d2 == 0 and nb >= 2 * d2

    @functools.partial(
        pl.kernel,
        out_type=[jax.ShapeDtypeStruct((n_nodes, feat), jnp.float32),
                  jax.ShapeDtypeStruct((n_nodes, feat), jnp.float32)],
        mesh=mesh,
        scratch_types=[
            pltpu.VMEM((d2, _K), jnp.int32),
            pltpu.VMEM((d2, _K), jnp.int32),
            pltpu.VMEM((_D * _K, feat), jnp.float32),
            pltpu.VMEM_SHARED((n_nodes, feat), jnp.float32),
        ] + [pltpu.SemaphoreType.DMA] * (5 * _D),
    )
    def k(g_hbm, adj_hbm, o0_hbm, o1_hbm, src_v, dst_v, rows_v, acc, *sems):
        gsem = sems[:_D]
        ssem = sems[_D:3 * _D]
        dsem = sems[3 * _D:5 * _D]
        cid = lax.axis_index("c")
        sid = lax.axis_index("s")
        wid = sid * _NC + cid
        row0 = sid * rpt
        pltpu.sync_copy(g_hbm.at[pl.ds(row0, rpt)], acc.at[pl.ds(row0, rpt)])
        plsc.subcore_barrier()

        def slot(d):
            return rows_v.at[pl.ds(d * _K, _K)]

        def fire_idx(b, e):
            pltpu.async_copy(adj_hbm.at[0, wid, b], src_v.at[e], ssem[e])
            pltpu.async_copy(adj_hbm.at[1, wid, b], dst_v.at[e], dsem[e])

        def wait_sidx(b, e):
            pltpu.make_async_copy(
                adj_hbm.at[0, wid, b], src_v.at[e], ssem[e]).wait()

        def fire_gather(b, f, d):
            pltpu.async_copy(g_hbm.at[src_v.at[f]], slot(d), gsem[d])

        def scatter(b, e, d):
            pltpu.make_async_copy(
                g_hbm.at[src_v.at[e]], slot(d), gsem[d]).wait()
            pltpu.make_async_copy(
                adj_hbm.at[1, wid, b], dst_v.at[e], dsem[e]).wait()
            pltpu.sync_copy(slot(d), acc.at[dst_v.at[e]], add=True)

        for e in range(d2):
            fire_idx(e, e)
        for d in range(_D):
            wait_sidx(d, d)
            fire_gather(d, d, d)

        def outer(gi, carry):
            for j in range(d2):
                b = gi * d2 + j
                d = j % _D
                f = (j + _D) % d2
                scatter(b, j, d)
                fire_idx(b + d2, j)
                wait_sidx(b + _D, f)
                fire_gather(b + _D, f, d)
            return carry

        lax.fori_loop(0, nb // d2 - 1, outer, 0)
        for j in range(_D):
            b = nb - d2 + j
            d = j % _D
            f = (j + _D) % d2
            scatter(b, j, d)
            wait_sidx(b + _D, f)
            fire_gather(b + _D, f, d)
        for j in range(_D, d2):
            scatter(nb - d2 + j, j, j % _D)

        plsc.subcore_barrier()

        @pl.when(cid == 0)
        def _():
            pltpu.sync_copy(acc.at[pl.ds(row0, rpt)],
                            o0_hbm.at[pl.ds(row0, rpt)])

        @pl.when(cid == 1)
        def _():
            pltpu.sync_copy(acc.at[pl.ds(row0, rpt)],
                            o1_hbm.at[pl.ds(row0, rpt)])

    return k(g, adjr)


_BR = 2000


def _tc_layer1(x, W1, p0, p1, n_pad):
    n, fin = x.shape
    h = W1.shape[1]

    def body(x_ref, w_ref, p0_ref, p1_ref, g_ref, s_ref):
        deg = p0_ref[:, 0:1] + p1_ref[:, 0:1] - 1.0
        s = lax.rsqrt(deg)
        hv = jnp.dot(x_ref[...], w_ref[...], preferred_element_type=jnp.float32)
        g_ref[...] = s * hv
        s_ref[...] = jnp.broadcast_to(s, (_BR, 16))

    return pl.pallas_call(
        body,
        grid=(n // _BR,),
        in_specs=[
            pl.BlockSpec((_BR, fin), lambda i: (i, 0)),
            pl.BlockSpec((fin, h), lambda i: (0, 0)),
            pl.BlockSpec((_BR, 16), lambda i: (i, 0)),
            pl.BlockSpec((_BR, 16), lambda i: (i, 0)),
        ],
        out_specs=[
            pl.BlockSpec((_BR, h), lambda i: (i, 0)),
            pl.BlockSpec((_BR, 16), lambda i: (i, 0)),
        ],
        out_shape=[
            jax.ShapeDtypeStruct((n_pad, h), jnp.float32),
            jax.ShapeDtypeStruct((n_pad, 16), jnp.float32),
        ],
    )(x, W1, p0, p1)


def _tc_layer2(a0, a1, g1, s, W2, b1, n_real):
    n_pad, h = g1.shape
    c = W2.shape[1]

    def body(a0_ref, a1_ref, g1_ref, s_ref, w_ref, b_ref, out_ref):
        sv = s_ref[:, 0:1]
        pre = sv * (a0_ref[...] + a1_ref[...] - g1_ref[...]) + b_ref[...]
        hv = jnp.maximum(pre, 0.0)
        h2 = jnp.dot(hv, w_ref[...], preferred_element_type=jnp.float32)
        out_ref[...] = sv * h2

    return pl.pallas_call(
        body,
        grid=(n_real // _BR,),
        in_specs=[
            pl.BlockSpec((_BR, h), lambda i: (i, 0)),
            pl.BlockSpec((_BR, h), lambda i: (i, 0)),
            pl.BlockSpec((_BR, h), lambda i: (i, 0)),
            pl.BlockSpec((_BR, 16), lambda i: (i, 0)),
            pl.BlockSpec((h, c), lambda i: (0, 0)),
            pl.BlockSpec((1, h), lambda i: (0, 0)),
        ],
        out_specs=pl.BlockSpec((_BR, c), lambda i: (i, 0)),
        out_shape=jax.ShapeDtypeStruct((n_pad, c), jnp.float32),
    )(a0, a1, g1, s, W2, b1)


def _tc_out(c0, c1, g2, s, b2, n_real):
    cp = g2.shape[1]
    c = b2.shape[1]

    def body(c0_ref, c1_ref, g2_ref, s_ref, b_ref, out_ref):
        sv = s_ref[:, 0:1]
        zc0 = c0_ref[:, 0:c]
        zc1 = c1_ref[:, 0:c]
        zg2 = g2_ref[:, 0:c]
        z = sv * (zc0 + zc1 - zg2) + b_ref[...]
        z = z - jnp.max(z, axis=1, keepdims=True)
        out_ref[...] = z - jnp.log(jnp.sum(jnp.exp(z), axis=1, keepdims=True))

    return pl.pallas_call(
        body,
        grid=(n_real // _BR,),
        in_specs=[
            pl.BlockSpec((_BR, cp), lambda i: (i, 0)),
            pl.BlockSpec((_BR, cp), lambda i: (i, 0)),
            pl.BlockSpec((_BR, cp), lambda i: (i, 0)),
            pl.BlockSpec((_BR, 16), lambda i: (i, 0)),
            pl.BlockSpec((1, c), lambda i: (0, 0)),
        ],
        out_specs=pl.BlockSpec((_BR, c), lambda i: (i, 0)),
        out_shape=jax.ShapeDtypeStruct((n_real, c), jnp.float32),
    )(c0, c1, g2, s, b2)


def kernel(x, adj, W1, b1, W2, b2):
    n = x.shape[0]
    n_pad = -(-n // 1280) * 1280
    e = adj.shape[1]
    nb = e // _NW // _K
    adjr = adj.astype(jnp.int32).reshape(2, _NW, nb, _K)
    ncls = W2.shape[1]
    W2p = jnp.pad(W2, ((0, 0), (0, 128 - ncls)))

    p0, p1 = _sc_degree_hist(adjr, n_pad)
    g1, s = _tc_layer1(x, W1, p0, p1, n_pad)
    a0, a1 = _sc_edge_agg(g1, adjr)
    g2 = _tc_layer2(a0, a1, g1, s, W2p, b1.reshape(1, -1), n)
    c0, c1 = _sc_edge_agg(g2, adjr)
    return _tc_out(c0, c1, g2, s, b2.reshape(1, -1), n)


# speedup vs baseline: 39.1056x; 1.0806x over previous
import functools

import jax
import jax.numpy as jnp
from jax import lax
from jax.experimental import pallas as pl
from jax.experimental.pallas import tpu as pltpu
from jax.experimental.pallas import tpu_sc as plsc

_NC = 2
_NS = 16
_NW = _NC * _NS
_K = 40
_D = 5
_W = 8


def _sc_degree_hist(adjr, n_nodes):
    nb = adjr.shape[2]
    rpt = n_nodes // _NS
    mesh = plsc.VectorSubcoreMesh(core_axis_name="c", subcore_axis_name="s")

    @functools.partial(
        pl.kernel,
        out_type=[jax.ShapeDtypeStruct((n_nodes, 16), jnp.float32),
                  jax.ShapeDtypeStruct((n_nodes, 16), jnp.float32)],
        mesh=mesh,
        scratch_types=[
            pltpu.VMEM((nb, _K), jnp.int32),
            pltpu.VMEM((_K, 16), jnp.float32),
            pltpu.VMEM_SHARED((n_nodes, 16), jnp.float32),
            pltpu.SemaphoreType.DMA,
        ],
    )
    def k(adj_hbm, p0_hbm, p1_hbm, idx_v, ones_v, acc, sem):
        cid = lax.axis_index("c")
        sid = lax.axis_index("s")
        wid = sid * _NC + cid
        row0 = sid * rpt
        pltpu.sync_copy(adj_hbm.at[1, wid], idx_v)

        def setone(r, carry):
            ones_v[r, :] = jnp.full((16,), 1.0, jnp.float32)
            return carry

        lax.fori_loop(0, _K, setone, 0)

        def init(j, carry):
            pltpu.sync_copy(ones_v, acc.at[pl.ds(row0 + j * _K, _K)])
            return carry

        lax.fori_loop(0, rpt // _K, init, 0)
        plsc.subcore_barrier()

        def fire(b, carry):
            pltpu.async_copy(ones_v, acc.at[idx_v.at[b]], sem, add=True)
            return carry

        def fire_and_wait(b, carry):
            pltpu.async_copy(ones_v, acc.at[idx_v.at[b]], sem, add=True)
            pltpu.make_async_copy(ones_v, acc.at[idx_v.at[0]], sem).wait()
            return carry

        def drain(b, carry):
            pltpu.make_async_copy(ones_v, acc.at[idx_v.at[0]], sem).wait()
            return carry

        lax.fori_loop(0, _W, fire, 0)
        lax.fori_loop(_W, nb, fire_and_wait, 0)
        lax.fori_loop(0, _W, drain, 0)
        plsc.subcore_barrier()

        @pl.when(cid == 0)
        def _():
            pltpu.sync_copy(acc.at[pl.ds(row0, rpt)],
                            p0_hbm.at[pl.ds(row0, rpt)])

        @pl.when(cid == 1)
        def _():
            pltpu.sync_copy(acc.at[pl.ds(row0, rpt)],
                            p1_hbm.at[pl.ds(row0, rpt)])

    return k(adjr)


def _sc_edge_agg(g, adjr, tc_tiling=True):
    n_nodes, feat = g.shape
    nb = adjr.shape[2]
    rpt = n_nodes // _NS
    mesh = plsc.VectorSubcoreMesh(core_axis_name="c", subcore_axis_name="s")
    d2 = 2 * _D
    assert nb % d2 == 0 and nb >= 2 * d2

    @functools.partial(
        pl.kernel,
        out_type=[jax.ShapeDtypeStruct((n_nodes, feat), jnp.float32),
                  jax.ShapeDtypeStruct((n_nodes, feat), jnp.float32)],
        mesh=mesh,
        compiler_params=pltpu.CompilerParams(use_tc_tiling_on_sc=tc_tiling),
        scratch_types=[
            pltpu.VMEM((d2, _K), jnp.int32),
            pltpu.VMEM((d2, _K), jnp.int32),
            pltpu.VMEM((_D * _K, feat), jnp.float32),
            pltpu.VMEM_SHARED((n_nodes, feat), jnp.float32),
        ] + [pltpu.SemaphoreType.DMA] * (5 * _D),
    )
    def k(g_hbm, adj_hbm, o0_hbm, o1_hbm, src_v, dst_v, rows_v, acc, *sems):
        gsem = sems[:_D]
        ssem = sems[_D:3 * _D]
        dsem = sems[3 * _D:5 * _D]
        cid = lax.axis_index("c")
        sid = lax.axis_index("s")
        wid = sid * _NC + cid
        row0 = sid * rpt
        pltpu.sync_copy(g_hbm.at[pl.ds(row0, rpt)], acc.at[pl.ds(row0, rpt)])
        plsc.subcore_barrier()

        def slot(d):
            return rows_v.at[pl.ds(d * _K, _K)]

        def fire_idx(b, e):
            pltpu.async_copy(adj_hbm.at[0, wid, b], src_v.at[e], ssem[e])
            pltpu.async_copy(adj_hbm.at[1, wid, b], dst_v.at[e], dsem[e])

        def wait_sidx(b, e):
            pltpu.make_async_copy(
                adj_hbm.at[0, wid, b], src_v.at[e], ssem[e]).wait()

        def fire_gather(b, f, d):
            pltpu.async_copy(g_hbm.at[src_v.at[f]], slot(d), gsem[d])

        def scatter(b, e, d):
            pltpu.make_async_copy(
                g_hbm.at[src_v.at[e]], slot(d), gsem[d]).wait()
            pltpu.make_async_copy(
                adj_hbm.at[1, wid, b], dst_v.at[e], dsem[e]).wait()
            pltpu.sync_copy(slot(d), acc.at[dst_v.at[e]], add=True)

        for e in range(d2):
            fire_idx(e, e)
        for d in range(_D):
            wait_sidx(d, d)
            fire_gather(d, d, d)

        def outer(gi, carry):
            for j in range(d2):
                b = gi * d2 + j
                d = j % _D
                f = (j + _D) % d2
                scatter(b, j, d)
                fire_idx(b + d2, j)
                wait_sidx(b + _D, f)
                fire_gather(b + _D, f, d)
            return carry

        lax.fori_loop(0, nb // d2 - 1, outer, 0)
        for j in range(_D):
            b = nb - d2 + j
            d = j % _D
            f = (j + _D) % d2
            scatter(b, j, d)
            wait_sidx(b + _D, f)
            fire_gather(b + _D, f, d)
        for j in range(_D, d2):
            scatter(nb - d2 + j, j, j % _D)

        plsc.subcore_barrier()

        @pl.when(cid == 0)
        def _():
            pltpu.sync_copy(acc.at[pl.ds(row0, rpt)],
                            o0_hbm.at[pl.ds(row0, rpt)])

        @pl.when(cid == 1)
        def _():
            pltpu.sync_copy(acc.at[pl.ds(row0, rpt)],
                            o1_hbm.at[pl.ds(row0, rpt)])

    return k(g, adjr)


_BR = 2000


def _tc_layer1(x, W1, p0, p1, n_pad):
    n, fin = x.shape
    h = W1.shape[1]

    def body(x_ref, w_ref, p0_ref, p1_ref, g_ref, s_ref):
        deg = p0_ref[:, 0:1] + p1_ref[:, 0:1] - 1.0
        s = lax.rsqrt(deg)
        hv = jnp.dot(x_ref[...], w_ref[...], preferred_element_type=jnp.float32)
        g_ref[...] = s * hv
        s_ref[...] = jnp.broadcast_to(s, (_BR, 16))

    return pl.pallas_call(
        body,
        grid=(n // _BR,),
        in_specs=[
            pl.BlockSpec((_BR, fin), lambda i: (i, 0)),
            pl.BlockSpec((fin, h), lambda i: (0, 0)),
            pl.BlockSpec((_BR, 16), lambda i: (i, 0)),
            pl.BlockSpec((_BR, 16), lambda i: (i, 0)),
        ],
        out_specs=[
            pl.BlockSpec((_BR, h), lambda i: (i, 0)),
            pl.BlockSpec((_BR, 16), lambda i: (i, 0)),
        ],
        out_shape=[
            jax.ShapeDtypeStruct((n_pad, h), jnp.float32),
            jax.ShapeDtypeStruct((n_pad, 16), jnp.float32),
        ],
    )(x, W1, p0, p1)


def _tc_layer2(a0, a1, g1, s, W2, b1, n_real):
    n_pad, h = g1.shape
    c = W2.shape[1]

    def body(a0_ref, a1_ref, g1_ref, s_ref, w_ref, b_ref, out_ref):
        sv = s_ref[:, 0:1]
        pre = sv * (a0_ref[...] + a1_ref[...] - g1_ref[...]) + b_ref[...]
        hv = jnp.maximum(pre, 0.0)
        h2 = jnp.dot(hv, w_ref[...], preferred_element_type=jnp.float32)
        out_ref[...] = sv * h2

    return pl.pallas_call(
        body,
        grid=(n_real // _BR,),
        in_specs=[
            pl.BlockSpec((_BR, h), lambda i: (i, 0)),
            pl.BlockSpec((_BR, h), lambda i: (i, 0)),
            pl.BlockSpec((_BR, h), lambda i: (i, 0)),
            pl.BlockSpec((_BR, 16), lambda i: (i, 0)),
            pl.BlockSpec((h, c), lambda i: (0, 0)),
            pl.BlockSpec((1, h), lambda i: (0, 0)),
        ],
        out_specs=pl.BlockSpec((_BR, c), lambda i: (i, 0)),
        out_shape=jax.ShapeDtypeStruct((n_pad, c), jnp.float32),
    )(a0, a1, g1, s, W2, b1)


def _tc_out(c0, c1, g2, s, b2, n_real):
    cp = g2.shape[1]
    c = b2.shape[1]

    def body(c0_ref, c1_ref, g2_ref, s_ref, b_ref, out_ref):
        sv = s_ref[:, 0:1]
        zc0 = c0_ref[:, 0:c]
        zc1 = c1_ref[:, 0:c]
        zg2 = g2_ref[:, 0:c]
        z = sv * (zc0 + zc1 - zg2) + b_ref[...]
        z = z - jnp.max(z, axis=1, keepdims=True)
        out_ref[...] = z - jnp.log(jnp.sum(jnp.exp(z), axis=1, keepdims=True))

    return pl.pallas_call(
        body,
        grid=(n_real // _BR,),
        in_specs=[
            pl.BlockSpec((_BR, cp), lambda i: (i, 0)),
            pl.BlockSpec((_BR, cp), lambda i: (i, 0)),
            pl.BlockSpec((_BR, cp), lambda i: (i, 0)),
            pl.BlockSpec((_BR, 16), lambda i: (i, 0)),
            pl.BlockSpec((1, c), lambda i: (0, 0)),
        ],
        out_specs=pl.BlockSpec((_BR, c), lambda i: (i, 0)),
        out_shape=jax.ShapeDtypeStruct((n_real, c), jnp.float32),
    )(c0, c1, g2, s, b2)


def kernel(x, adj, W1, b1, W2, b2):
    n = x.shape[0]
    n_pad = -(-n // 1280) * 1280
    e = adj.shape[1]
    nb = e // _NW // _K
    adjr = adj.astype(jnp.int32).reshape(2, _NW, nb, _K)
    p0, p1 = _sc_degree_hist(adjr, n_pad)
    g1, s = _tc_layer1(x, W1, p0, p1, n_pad)
    a0, a1 = _sc_edge_agg(g1, adjr)
    g2 = _tc_layer2(a0, a1, g1, s, W2, b1.reshape(1, -1), n)
    c0, c1 = _sc_edge_agg(g2, adjr, tc_tiling=False)
    return _tc_out(c0, c1, g2, s, b2.reshape(1, -1), n)
